# baseline (speedup 1.0000x reference)
import functools

import jax
import jax.numpy as jnp
from jax import lax
from jax.experimental import pallas as pl
from jax.experimental.pallas import tpu as pltpu
from jax.experimental.pallas import tpu_sc as plsc

B, P, N, C = 2, 12000, 32, 9
U = 64
H, WIDTH = 496, 432
HW = H * WIDTH
EPS = 1e-3
R = P * N
NTILES = 32
CPT = B * HW // NTILES
SEG = 128
UP = 128
NU = 8
PAD_ROWS = 8
CPTP = 13440
GF = 24064
LIST_CAP = CPT + 16


def _pfn_body(f_ref, wt_ref, gam_ref, bet_ref, out_ref, acc_ref, q_ref, s_ref):
    n_idx = pl.program_id(1)
    wt = wt_ref[...]
    f8 = f_ref[0]
    xs = [lax.dot_general(f8[:, j, :], wt, (((0,), (0,)), ((), ())),
                          preferred_element_type=jnp.float32)
          for j in range(NU)]
    while len(xs) > 1:
        xs = [jnp.maximum(xs[2 * i], xs[2 * i + 1]) for i in range(len(xs) // 2)]
    xn = xs[0]

    f2 = f8.reshape(C * NU, P)
    q72 = lax.dot_general(f2, f2, (((1,), (1,)), ((), ())),
                          preferred_element_type=jnp.float32)
    s72 = lax.dot_general(jnp.ones((1, P), jnp.float32), f2,
                          (((1,), (1,)), ((), ())),
                          preferred_element_type=jnp.float32)
    r72 = lax.broadcasted_iota(jnp.int32, (C * NU, 1), 0)
    sel = (r72 // NU == lax.broadcasted_iota(jnp.int32, (1, C), 1)
           ).astype(jnp.float32)
    dmask = (r72 % NU == lax.broadcasted_iota(jnp.int32, (1, C * NU), 1) % NU
             ).astype(jnp.float32)
    qp = lax.dot_general(
        lax.dot_general(q72 * dmask, sel, (((1,), (0,)), ((), ())),
                        preferred_element_type=jnp.float32),
        sel, (((0,), (0,)), ((), ())),
        preferred_element_type=jnp.float32)
    sp = lax.dot_general(s72, sel, (((1,), (0,)), ((), ())),
                         preferred_element_type=jnp.float32)

    @pl.when(n_idx == 0)
    def _():
        acc_ref[...] = xn
        q_ref[...] = qp
        s_ref[...] = sp

    @pl.when(n_idx > 0)
    def _():
        acc_ref[...] = jnp.maximum(acc_ref[...], xn)
        q_ref[...] += qp
        s_ref[...] += sp

    @pl.when(n_idx == N // NU - 1)
    def _():
        nn = jnp.float32(R)
        mu = lax.dot_general(s_ref[...] / nn, wt, (((1,), (0,)), ((), ())),
                             preferred_element_type=jnp.float32)
        aw = lax.dot_general(q_ref[...] / nn, wt, (((1,), (0,)), ((), ())),
                             preferred_element_type=jnp.float32)
        ex2 = jnp.sum(aw * wt, axis=0, keepdims=True)
        var = ex2 - mu * mu
        scale = gam_ref[...] * lax.rsqrt(var + EPS)
        bias = bet_ref[...] - mu * scale
        out_ref[0] = jnp.maximum(acc_ref[...] * scale + bias, 0.0)


def _pfn(ft, wt, gamma2, beta2):
    return pl.pallas_call(
        _pfn_body,
        grid=(B, N // NU),
        in_specs=[
            pl.BlockSpec((1, C, NU, P), lambda b, n: (b, 0, n, 0)),
            pl.BlockSpec((C, UP), lambda b, n: (0, 0)),
            pl.BlockSpec((1, UP), lambda b, n: (0, 0)),
            pl.BlockSpec((1, UP), lambda b, n: (0, 0)),
        ],
        out_specs=pl.BlockSpec((1, P, UP), lambda b, n: (b, 0, 0)),
        out_shape=jax.ShapeDtypeStruct((B, P, UP), jnp.float32),
        scratch_shapes=[pltpu.VMEM((P, UP), jnp.float32),
                        pltpu.VMEM((C, C), jnp.float32),
                        pltpu.VMEM((1, C), jnp.float32)],
        compiler_params=pltpu.CompilerParams(
            vmem_limit_bytes=100 * 1024 * 1024),
    )(ft, wt, gamma2, beta2)


XB = 8


def _emit_body(ct_ref, wm_ref, out_ref):
    for xi in range(XB):
        x = ct_ref[xi * H:(xi + 1) * H, :U]
        keep = wm_ref[0, 0, xi * H:(xi + 1) * H] >= 0
        out_ref[0, :, xi, :] = jnp.where(keep[None, :], x.T, 0.0)


def _emit(canvas_t, wmap):
    wmap3 = wmap.reshape(B * (WIDTH // XB), 1, XB * H)
    return pl.pallas_call(
        _emit_body,
        grid=(B, WIDTH // XB),
        in_specs=[
            pl.BlockSpec((XB * H, UP), lambda b, j: (b * (WIDTH // XB) + j, 0)),
            pl.BlockSpec((1, 1, XB * H),
                         lambda b, j: (b * (WIDTH // XB) + j, 0, 0)),
        ],
        out_specs=pl.BlockSpec((1, U, XB, H), lambda b, j: (b, 0, j, 0)),
        out_shape=jax.ShapeDtypeStruct((B, U, WIDTH, H), jnp.float32),
    )(canvas_t, wmap3)


_MESH = plsc.VectorSubcoreMesh(core_axis_name="c", subcore_axis_name="s")
_SC_PARAMS = pltpu.CompilerParams(needs_layout_passes=False)


def _sc_wmap(gflat):
    @functools.partial(
        pl.kernel,
        out_type=jax.ShapeDtypeStruct((NTILES, CPTP), jnp.int32),
        mesh=_MESH,
        scratch_types=[
            pltpu.VMEM((GF,), jnp.int32),
            pltpu.VMEM((CPTP + 16,), jnp.int32),
        ],
        compiler_params=_SC_PARAMS,
    )
    def body(gflat_hbm, wmap_hbm, gflat_v, wmap_v):
        cid = lax.axis_index("c")
        sid = lax.axis_index("s")
        wid = sid * 2 + cid
        lo = wid * CPT
        pltpu.sync_copy(gflat_hbm, gflat_v)
        lane = lax.iota(jnp.int32, 16)
        neg1 = jnp.full((16,), -1, jnp.int32)

        def init_w(i, _):
            wmap_v[pl.ds(i * 16, 16)] = neg1
            return 0
        lax.fori_loop(0, (CPTP + 16) // 16, init_w, 0)

        def chunk(i, _):
            f = gflat_v[pl.ds(i * 16, 16)]
            p = lane + i * 16
            m = (f >= lo) & (f < lo + CPT)
            fc = jnp.where(m, f - lo, CPT + lane)
            for l in range(16):
                plsc.store_scatter(wmap_v, [fc], p, mask=m & (lane == l))
            return 0
        lax.fori_loop(0, GF // 16, chunk, 0)

        pltpu.sync_copy(wmap_v.at[pl.ds(0, CPTP)], wmap_hbm.at[wid])

    return body(gflat)


def _sc_move(wmap2d, xmax_rows):
    @functools.partial(
        pl.kernel,
        out_type=jax.ShapeDtypeStruct((B * HW + PAD_ROWS, UP), jnp.float32),
        mesh=_MESH,
        scratch_types=[
            pltpu.VMEM((CPTP,), jnp.int32),
            pltpu.VMEM((LIST_CAP,), jnp.int32),
            pltpu.VMEM((LIST_CAP,), jnp.int32),
            pltpu.VMEM((1, SEG), jnp.int32),
            pltpu.VMEM((1, SEG), jnp.int32),
            pltpu.VMEM((SEG, UP), jnp.float32),
        ],
        compiler_params=_SC_PARAMS,
    )
    def body(wmap_hbm, xmax_hbm, canvas_hbm,
             wmap_v, plist_v, flist_v, pseg_v, fseg_v, rows_v):
        cid = lax.axis_index("c")
        sid = lax.axis_index("s")
        wid = sid * 2 + cid
        lo = wid * CPT
        pltpu.sync_copy(wmap_hbm.at[wid], wmap_v)
        lane = lax.iota(jnp.int32, 16)
        padrow = jnp.full((16,), B * HW, jnp.int32) + (lane & (PAD_ROWS - 1))
        zero16 = jnp.zeros((16,), jnp.int32)

        def init_l(i, _):
            plist_v[pl.ds(i * 16, 16)] = zero16
            flist_v[pl.ds(i * 16, 16)] = padrow
            return 0
        lax.fori_loop(0, LIST_CAP // 16, init_l, 0)

        def cells(i, cur):
            w = wmap_v[pl.ds(i * 16, 16)]
            m = w >= 0
            plsc.store_compressed(plist_v.at[pl.ds(cur, 16)], w, mask=m)
            plsc.store_compressed(flist_v.at[pl.ds(cur, 16)],
                                  lane + (lo + i * 16), mask=m)
            return cur + jnp.sum(m.astype(jnp.int32))
        nwin = lax.fori_loop(0, CPT // 16, cells, 0)
        nseg = (nwin + SEG - 1) // SEG

        def seg(s, _):
            def cp(j, _):
                pseg_v[0, pl.ds(j * 16, 16)] = plist_v[pl.ds(s * SEG + j * 16, 16)]
                fseg_v[0, pl.ds(j * 16, 16)] = flist_v[pl.ds(s * SEG + j * 16, 16)]
                return 0
            lax.fori_loop(0, SEG // 16, cp, 0)
            pltpu.sync_copy(xmax_hbm.at[pseg_v.at[0]], rows_v)
            pltpu.sync_copy(rows_v, canvas_hbm.at[fseg_v.at[0]])
            return 0
        lax.fori_loop(0, nseg, seg, 0)

    return body(wmap2d, xmax_rows)


def kernel(feats, coords, W, gamma, beta):
    wt = jnp.pad(W.T, ((0, 0), (0, UP - U)))
    gamma2 = jnp.pad(gamma.reshape(1, U), ((0, 0), (0, UP - U)))
    beta2 = jnp.pad(beta.reshape(1, U), ((0, 0), (0, UP - U)))

    ft = jnp.transpose(feats, (0, 3, 2, 1))
    xmax = _pfn(ft, wt, gamma2, beta2)
    xmax_rows = xmax.reshape(B * P, UP)

    c = coords.astype(jnp.int32)
    gflat = (c[:, :, 1] * H + c[:, :, 0]
             + (jnp.arange(B, dtype=jnp.int32) * HW)[:, None]).reshape(B * P)
    gflat = jnp.pad(gflat, (0, GF - B * P), constant_values=jnp.int32(2 ** 29))

    wmap2d = _sc_wmap(gflat)
    canvas_t = _sc_move(wmap2d, xmax_rows)
    wmap = wmap2d[:, :CPT].reshape(B * HW)
    out = _emit(canvas_t, wmap)
    return out.swapaxes(2, 3)
